# Initial kernel scaffold; baseline (speedup 1.0000x reference)
import jax
import jax.numpy as jnp
from jax.experimental import pallas as pl


def kernel(x, weight):
    raise NotImplementedError("write your pallas kernel here")


# speedup vs baseline: 1.0254x; 1.0254x over previous
import jax
import jax.numpy as jnp
from jax.experimental import pallas as pl
from jax.experimental.pallas import tpu as pltpu
from jax.experimental.pallas import tpu_sc as plsc

N = 16384
K = 8192
KH = K // 2
D = 32
COMMIT = 0.25

BN = 256
NB = N // BN
GW = 128


def _half_min(v, base):
    mv = jnp.min(v, axis=1, keepdims=True)
    kidx = jax.lax.broadcasted_iota(jnp.int32, (BN, KH), 1) + base
    idx = jnp.min(jnp.where(v == mv, kidx, K), axis=1)
    return mv, idx


def _argmin_body(x_ref, wt_ref, x2_ref, w2_ref, idx_ref, loss_ref):
    i = pl.program_id(0)
    xb = x_ref[...].astype(jnp.bfloat16)
    s = jax.lax.dot_general(
        xb, wt_ref[...], (((1,), (0,)), ((), ())),
        preferred_element_type=jnp.float32)
    score = (x2_ref[...] + w2_ref[...]) - 2.0 * s
    v = jnp.sqrt(jnp.maximum(score, 0.0))
    m0, i0 = _half_min(v[:, :KH], 0)
    m1, i1 = _half_min(v[:, KH:], KH)
    m0b = m0.astype(jnp.bfloat16).astype(jnp.float32)
    pick = m1 < m0b
    idx_ref[0, 0, :] = jnp.where(pick[:, 0], i1, i0)

    dmin = jnp.where(pick, m1 * m1, m0 * m0)
    part = jnp.sum(dmin, keepdims=True).reshape(1, 1)
    acc = jnp.where(i == 0, part, loss_ref[...] + part)
    acc = jnp.where(i == NB - 1, acc * ((1.0 + COMMIT) / (N * D)), acc)
    loss_ref[...] = acc


def _argmin_call(x, wt, x2, w2):
    return pl.pallas_call(
        _argmin_body,
        grid=(NB,),
        in_specs=[
            pl.BlockSpec((BN, D), lambda i: (i, 0)),
            pl.BlockSpec((D, K), lambda i: (0, 0)),
            pl.BlockSpec((BN, 1), lambda i: (i, 0)),
            pl.BlockSpec((1, K), lambda i: (0, 0)),
        ],
        out_specs=[
            pl.BlockSpec((1, 1, BN), lambda i: (i, 0, 0)),
            pl.BlockSpec((1, 1), lambda i: (0, 0)),
        ],
        out_shape=[
            jax.ShapeDtypeStruct((NB, 1, BN), jnp.int32),
            jax.ShapeDtypeStruct((1, 1), jnp.float32),
        ],
    )(x, wt, x2, w2)


def _sc_gather(weight_pad, idx_row):
    mesh = plsc.VectorSubcoreMesh(core_axis_name="c", subcore_axis_name="s")

    @pl.kernel(out_type=jax.ShapeDtypeStruct((N, 128), jnp.float32), mesh=mesh)
    def gather_kernel(w_hbm, i_hbm, o_hbm):
        def body(i_vmem, o_vmem):
            pltpu.sync_copy(w_hbm.at[i_vmem.at[0]], o_vmem)

        pltpu.emit_pipeline(
            body,
            grid=(N // GW,),
            in_specs=[pl.BlockSpec((1, GW), index_map=lambda i: (0, i))],
            out_specs=[pl.BlockSpec((GW, 128), index_map=lambda i: (i, 0))],
            core_axis_name=("c", "s"),
            dimension_semantics=(pltpu.PARALLEL,),
        )(i_hbm, o_hbm)

    return gather_kernel(weight_pad, idx_row)


def kernel(x, weight):
    x2 = jnp.sum(x * x, axis=-1, keepdims=True)
    w2 = jnp.sum(weight * weight, axis=-1)[None, :]
    wt = weight.T.astype(jnp.bfloat16)
    idx3, loss = _argmin_call(x, wt, x2, w2)
    idx = idx3.reshape(N)
    weight_pad = jnp.pad(weight, ((0, 0), (0, 128 - D)))
    quantised = _sc_gather(weight_pad, idx.reshape(1, N))[:, :D]
    return (quantised, loss.reshape(()), idx)


# speedup vs baseline: 1.2467x; 1.2158x over previous
import jax
import jax.numpy as jnp
from jax.experimental import pallas as pl
from jax.experimental.pallas import tpu as pltpu
from jax.experimental.pallas import tpu_sc as plsc

N = 16384
K = 8192
KH = K // 2
D = 32
COMMIT = 0.25

BN = 256
NB = N // BN
GW = 128


def _sqrt_preimage_hi(v):
    c = v * v
    u = jax.lax.bitcast_convert_type(c, jnp.int32)
    hi = jnp.full_like(c, -jnp.inf)
    for j in range(-3, 4):
        cj = jax.lax.bitcast_convert_type(u + j, jnp.float32)
        ok = (jnp.sqrt(cj) == v) & (cj > hi)
        hi = jnp.where(ok, cj, hi)
    return hi


def _half_min(c, base):
    m = jnp.min(c, axis=1, keepdims=True)
    v = jnp.sqrt(m)
    hi = _sqrt_preimage_hi(v)
    kidx = jax.lax.broadcasted_iota(jnp.int32, (BN, KH), 1) + base
    idx = jnp.min(jnp.where(c <= hi, kidx, K), axis=1)
    return m, v, idx


def _argmin_body(x_ref, wt_ref, x2_ref, w2_ref, idx_ref, loss_ref):
    i = pl.program_id(0)
    xb = x_ref[...].astype(jnp.bfloat16)
    s = jax.lax.dot_general(
        xb, wt_ref[...], (((1,), (0,)), ((), ())),
        preferred_element_type=jnp.float32)
    d2 = (x2_ref[...] + w2_ref[...]) + s
    c = jnp.maximum(d2, 0.0)
    m0, v0, i0 = _half_min(c[:, :KH], 0)
    m1, v1, i1 = _half_min(c[:, KH:], KH)
    v0b = v0.astype(jnp.bfloat16).astype(jnp.float32)
    pick = v1 < v0b
    idx_ref[0, 0, :] = jnp.where(pick[:, 0], i1, i0)

    dmin = jnp.where(pick, m1, m0)
    part = jnp.sum(dmin, keepdims=True).reshape(1, 1)
    acc = jnp.where(i == 0, part, loss_ref[...] + part)
    acc = jnp.where(i == NB - 1, acc * ((1.0 + COMMIT) / (N * D)), acc)
    loss_ref[...] = acc


def _argmin_call(x, wt, x2, w2):
    return pl.pallas_call(
        _argmin_body,
        grid=(NB,),
        in_specs=[
            pl.BlockSpec((BN, D), lambda i: (i, 0)),
            pl.BlockSpec((D, K), lambda i: (0, 0)),
            pl.BlockSpec((BN, 1), lambda i: (i, 0)),
            pl.BlockSpec((1, K), lambda i: (0, 0)),
        ],
        out_specs=[
            pl.BlockSpec((1, 1, BN), lambda i: (i, 0, 0)),
            pl.BlockSpec((1, 1), lambda i: (0, 0)),
        ],
        out_shape=[
            jax.ShapeDtypeStruct((NB, 1, BN), jnp.int32),
            jax.ShapeDtypeStruct((1, 1), jnp.float32),
        ],
    )(x, wt, x2, w2)


def _sc_gather(weight_pad, idx_row):
    mesh = plsc.VectorSubcoreMesh(core_axis_name="c", subcore_axis_name="s")

    @pl.kernel(out_type=jax.ShapeDtypeStruct((N, 128), jnp.float32), mesh=mesh)
    def gather_kernel(w_hbm, i_hbm, o_hbm):
        def body(i_vmem, o_vmem):
            pltpu.sync_copy(w_hbm.at[i_vmem.at[0]], o_vmem)

        pltpu.emit_pipeline(
            body,
            grid=(N // GW,),
            in_specs=[pl.BlockSpec((1, GW), index_map=lambda i: (0, i))],
            out_specs=[pl.BlockSpec((GW, 128), index_map=lambda i: (i, 0))],
            core_axis_name=("c", "s"),
            dimension_semantics=(pltpu.PARALLEL,),
        )(i_hbm, o_hbm)

    return gather_kernel(weight_pad, idx_row)


def kernel(x, weight):
    x2 = jnp.sum(x * x, axis=-1, keepdims=True)
    w2 = jnp.sum(weight * weight, axis=-1)[None, :]
    wt = (-2.0 * weight.T).astype(jnp.bfloat16)
    idx3, loss = _argmin_call(x, wt, x2, w2)
    idx = idx3.reshape(N)
    weight_pad = jnp.pad(weight, ((0, 0), (0, 128 - D)))
    quantised = _sc_gather(weight_pad, idx.reshape(1, N))[:, :D]
    return (quantised, loss.reshape(()), idx)


# speedup vs baseline: 1.3247x; 1.0625x over previous
import jax
import jax.numpy as jnp
from jax.experimental import pallas as pl
from jax.experimental.pallas import tpu as pltpu
from jax.experimental.pallas import tpu_sc as plsc

N = 16384
K = 8192
KH = K // 2
D = 32
COMMIT = 0.25

BN = 256
NB = N // BN
GW = 128


def _sqrt_preimage_hi(v):
    c = v * v
    u = jax.lax.bitcast_convert_type(c, jnp.int32)
    hi = jnp.full_like(c, -jnp.inf)
    for j in range(-3, 4):
        cj = jax.lax.bitcast_convert_type(u + j, jnp.float32)
        ok = (jnp.sqrt(cj) == v) & (cj > hi)
        hi = jnp.where(ok, cj, hi)
    return hi


def _half_min(d2h, base):
    m = jnp.maximum(jnp.min(d2h, axis=1, keepdims=True), 0.0)
    v = jnp.sqrt(m)
    hi = _sqrt_preimage_hi(v)
    kidx = jax.lax.broadcasted_iota(jnp.int32, (BN, KH), 1) + base
    idx = jnp.min(jnp.where(d2h <= hi, kidx, K), axis=1)
    return m, v, idx


def _argmin_body(x_ref, wt_ref, x2_ref, w2_ref, idx_ref, loss_ref):
    i = pl.program_id(0)
    xb = x_ref[...].astype(jnp.bfloat16)
    s = jax.lax.dot_general(
        xb, wt_ref[...], (((1,), (0,)), ((), ())),
        preferred_element_type=jnp.float32)
    d2 = (x2_ref[...] + w2_ref[...]) + s
    m0, v0, i0 = _half_min(d2[:, :KH], 0)
    m1, v1, i1 = _half_min(d2[:, KH:], KH)
    v0b = v0.astype(jnp.bfloat16).astype(jnp.float32)
    pick = v1 < v0b
    idx_ref[0, 0, :] = jnp.where(pick[:, 0], i1, i0)

    dmin = jnp.where(pick, m1, m0)
    part = jnp.sum(dmin, keepdims=True).reshape(1, 1)
    acc = jnp.where(i == 0, part, loss_ref[...] + part)
    acc = jnp.where(i == NB - 1, acc * ((1.0 + COMMIT) / (N * D)), acc)
    loss_ref[...] = acc


def _argmin_call(x, wt, x2, w2):
    return pl.pallas_call(
        _argmin_body,
        grid=(NB,),
        in_specs=[
            pl.BlockSpec((BN, D), lambda i: (i, 0)),
            pl.BlockSpec((D, K), lambda i: (0, 0)),
            pl.BlockSpec((BN, 1), lambda i: (i, 0)),
            pl.BlockSpec((1, K), lambda i: (0, 0)),
        ],
        out_specs=[
            pl.BlockSpec((1, 1, BN), lambda i: (i, 0, 0)),
            pl.BlockSpec((1, 1), lambda i: (0, 0)),
        ],
        out_shape=[
            jax.ShapeDtypeStruct((NB, 1, BN), jnp.int32),
            jax.ShapeDtypeStruct((1, 1), jnp.float32),
        ],
    )(x, wt, x2, w2)


def _sc_gather(weight_pad, idx_row):
    mesh = plsc.VectorSubcoreMesh(core_axis_name="c", subcore_axis_name="s")

    @pl.kernel(out_type=jax.ShapeDtypeStruct((N, 128), jnp.float32), mesh=mesh)
    def gather_kernel(w_hbm, i_hbm, o_hbm):
        def body(i_vmem, o_vmem):
            pltpu.sync_copy(w_hbm.at[i_vmem.at[0]], o_vmem)

        pltpu.emit_pipeline(
            body,
            grid=(N // GW,),
            in_specs=[pl.BlockSpec((1, GW), index_map=lambda i: (0, i))],
            out_specs=[pl.BlockSpec((GW, 128), index_map=lambda i: (i, 0))],
            core_axis_name=("c", "s"),
            dimension_semantics=(pltpu.PARALLEL,),
        )(i_hbm, o_hbm)

    return gather_kernel(weight_pad, idx_row)


def kernel(x, weight):
    x2 = jnp.sum(x * x, axis=-1, keepdims=True)
    w2 = jnp.sum(weight * weight, axis=-1)[None, :]
    wt = (-2.0 * weight.T).astype(jnp.bfloat16)
    idx3, loss = _argmin_call(x, wt, x2, w2)
    idx = idx3.reshape(N)
    weight_pad = jnp.pad(weight, ((0, 0), (0, 128 - D)))
    quantised = _sc_gather(weight_pad, idx.reshape(1, N))[:, :D]
    return (quantised, loss.reshape(()), idx)


# speedup vs baseline: 1.4832x; 1.1197x over previous
import jax
import jax.numpy as jnp
from jax.experimental import pallas as pl
from jax.experimental.pallas import tpu as pltpu
from jax.experimental.pallas import tpu_sc as plsc

N = 16384
K = 8192
KH = K // 2
D = 32
COMMIT = 0.25

BN = 256
NB = N // BN
GW = 128


def _sqrt_preimage_hi(v):
    c = v * v
    u = jax.lax.bitcast_convert_type(c, jnp.int32)
    hi = jnp.full_like(c, -jnp.inf)
    for j in range(-3, 4):
        cj = jax.lax.bitcast_convert_type(u + j, jnp.float32)
        ok = (jnp.sqrt(cj) == v) & (cj > hi)
        hi = jnp.where(ok, cj, hi)
    return hi


def _first_le(d2h, hi, kidx, base):
    idxf = jnp.min(jnp.where(d2h <= hi, kidx, float(K)), axis=1)
    return (idxf + base).astype(jnp.int32)


def _argmin_body(x_ref, wt_ref, x2_ref, w2_ref, kio_ref, idx_ref, loss_ref):
    i = pl.program_id(0)
    xb = x_ref[...].astype(jnp.bfloat16)
    s = jax.lax.dot_general(
        xb, wt_ref[...], (((1,), (0,)), ((), ())),
        preferred_element_type=jnp.float32)
    d2 = (x2_ref[...] + w2_ref[...]) + s
    m0r = jnp.min(d2[:, :KH], axis=1, keepdims=True)
    m1r = jnp.min(d2[:, KH:], axis=1, keepdims=True)
    M = jnp.maximum(jnp.concatenate([m0r, m1r], axis=1), 0.0)
    V = jnp.sqrt(M)
    HI = _sqrt_preimage_hi(V)
    kidx = kio_ref[...]
    i0 = _first_le(d2[:, :KH], HI[:, :1], kidx, 0)
    i1 = _first_le(d2[:, KH:], HI[:, 1:], kidx, KH)
    m0, m1 = M[:, :1], M[:, 1:]
    v0b = V[:, :1].astype(jnp.bfloat16).astype(jnp.float32)
    pick = V[:, 1:] < v0b
    idx_ref[0, 0, :] = jnp.where(pick[:, 0], i1, i0)

    dmin = jnp.where(pick, m1, m0)
    part = jnp.sum(dmin, keepdims=True).reshape(1, 1)
    acc = jnp.where(i == 0, part, loss_ref[...] + part)
    acc = jnp.where(i == NB - 1, acc * ((1.0 + COMMIT) / (N * D)), acc)
    loss_ref[...] = acc


def _argmin_call(x, wt, x2, w2, kio):
    return pl.pallas_call(
        _argmin_body,
        grid=(NB,),
        in_specs=[
            pl.BlockSpec((BN, D), lambda i: (i, 0)),
            pl.BlockSpec((D, K), lambda i: (0, 0)),
            pl.BlockSpec((BN, 1), lambda i: (i, 0)),
            pl.BlockSpec((1, K), lambda i: (0, 0)),
            pl.BlockSpec((1, KH), lambda i: (0, 0)),
        ],
        out_specs=[
            pl.BlockSpec((1, 1, BN), lambda i: (i, 0, 0)),
            pl.BlockSpec((1, 1), lambda i: (0, 0)),
        ],
        out_shape=[
            jax.ShapeDtypeStruct((NB, 1, BN), jnp.int32),
            jax.ShapeDtypeStruct((1, 1), jnp.float32),
        ],
    )(x, wt, x2, w2, kio)


def _sc_gather(weight_pad, idx_row):
    mesh = plsc.VectorSubcoreMesh(core_axis_name="c", subcore_axis_name="s")

    @pl.kernel(out_type=jax.ShapeDtypeStruct((N, 128), jnp.float32), mesh=mesh)
    def gather_kernel(w_hbm, i_hbm, o_hbm):
        def body(i_vmem, o_vmem):
            pltpu.sync_copy(w_hbm.at[i_vmem.at[0]], o_vmem)

        pltpu.emit_pipeline(
            body,
            grid=(N // GW,),
            in_specs=[pl.BlockSpec((1, GW), index_map=lambda i: (0, i))],
            out_specs=[pl.BlockSpec((GW, 128), index_map=lambda i: (i, 0))],
            core_axis_name=("c", "s"),
            dimension_semantics=(pltpu.PARALLEL,),
        )(i_hbm, o_hbm)

    return gather_kernel(weight_pad, idx_row)


def kernel(x, weight):
    x2 = jnp.sum(x * x, axis=-1, keepdims=True)
    w2 = jnp.sum(weight * weight, axis=-1)[None, :]
    wt = (-2.0 * weight.T).astype(jnp.bfloat16)
    kio = jnp.arange(KH, dtype=jnp.float32)[None, :]
    idx3, loss = _argmin_call(x, wt, x2, w2, kio)
    idx = idx3.reshape(N)
    weight_pad = jnp.pad(weight, ((0, 0), (0, 128 - D)))
    quantised = _sc_gather(weight_pad, idx.reshape(1, N))[:, :D]
    return (quantised, loss.reshape(()), idx)


# speedup vs baseline: 1.5379x; 1.0369x over previous
import jax
import jax.numpy as jnp
from jax.experimental import pallas as pl
from jax.experimental.pallas import tpu as pltpu
from jax.experimental.pallas import tpu_sc as plsc

N = 16384
K = 8192
KH = K // 2
D = 32
COMMIT = 0.25

BN = 512
NB = N // BN
GW = 128


def _sqrt_preimage_hi(v):
    c = v * v
    u = jax.lax.bitcast_convert_type(c, jnp.int32)
    hi = jnp.full_like(c, -jnp.inf)
    for j in range(-3, 4):
        cj = jax.lax.bitcast_convert_type(u + j, jnp.float32)
        ok = (jnp.sqrt(cj) == v) & (cj > hi)
        hi = jnp.where(ok, cj, hi)
    return hi


def _first_le(d2h, hi, kidx, base):
    idxf = jnp.min(jnp.where(d2h <= hi, kidx, float(K)), axis=1)
    return (idxf + base).astype(jnp.int32)


def _argmin_body(x_ref, wt_ref, x2_ref, w2_ref, kio_ref, idx_ref, loss_ref):
    i = pl.program_id(0)
    xb = x_ref[...].astype(jnp.bfloat16)
    s = jax.lax.dot_general(
        xb, wt_ref[...], (((1,), (0,)), ((), ())),
        preferred_element_type=jnp.float32)
    d2 = (x2_ref[...] + w2_ref[...]) + s
    m0r = jnp.min(d2[:, :KH], axis=1, keepdims=True)
    m1r = jnp.min(d2[:, KH:], axis=1, keepdims=True)
    M = jnp.maximum(jnp.concatenate([m0r, m1r], axis=1), 0.0)
    V = jnp.sqrt(M)
    HI = _sqrt_preimage_hi(V)
    kidx = kio_ref[...]
    i0 = _first_le(d2[:, :KH], HI[:, :1], kidx, 0)
    i1 = _first_le(d2[:, KH:], HI[:, 1:], kidx, KH)
    m0, m1 = M[:, :1], M[:, 1:]
    v0b = V[:, :1].astype(jnp.bfloat16).astype(jnp.float32)
    pick = V[:, 1:] < v0b
    idx_ref[0, 0, :] = jnp.where(pick[:, 0], i1, i0)

    dmin = jnp.where(pick, m1, m0)
    part = jnp.sum(dmin, keepdims=True).reshape(1, 1)
    acc = jnp.where(i == 0, part, loss_ref[...] + part)
    acc = jnp.where(i == NB - 1, acc * ((1.0 + COMMIT) / (N * D)), acc)
    loss_ref[...] = acc


def _argmin_call(x, wt, x2, w2, kio):
    return pl.pallas_call(
        _argmin_body,
        grid=(NB,),
        in_specs=[
            pl.BlockSpec((BN, D), lambda i: (i, 0)),
            pl.BlockSpec((D, K), lambda i: (0, 0)),
            pl.BlockSpec((BN, 1), lambda i: (i, 0)),
            pl.BlockSpec((1, K), lambda i: (0, 0)),
            pl.BlockSpec((1, KH), lambda i: (0, 0)),
        ],
        out_specs=[
            pl.BlockSpec((1, 1, BN), lambda i: (i, 0, 0)),
            pl.BlockSpec((1, 1), lambda i: (0, 0)),
        ],
        out_shape=[
            jax.ShapeDtypeStruct((NB, 1, BN), jnp.int32),
            jax.ShapeDtypeStruct((1, 1), jnp.float32),
        ],
    )(x, wt, x2, w2, kio)


def _sc_gather(weight_pad, idx_row):
    mesh = plsc.VectorSubcoreMesh(core_axis_name="c", subcore_axis_name="s")

    @pl.kernel(out_type=jax.ShapeDtypeStruct((N, 128), jnp.float32), mesh=mesh)
    def gather_kernel(w_hbm, i_hbm, o_hbm):
        def body(i_vmem, o_vmem):
            pltpu.sync_copy(w_hbm.at[i_vmem.at[0]], o_vmem)

        pltpu.emit_pipeline(
            body,
            grid=(N // GW,),
            in_specs=[pl.BlockSpec((1, GW), index_map=lambda i: (0, i))],
            out_specs=[pl.BlockSpec((GW, 128), index_map=lambda i: (i, 0))],
            core_axis_name=("c", "s"),
            dimension_semantics=(pltpu.PARALLEL,),
        )(i_hbm, o_hbm)

    return gather_kernel(weight_pad, idx_row)


def kernel(x, weight):
    x2 = jnp.sum(x * x, axis=-1, keepdims=True)
    w2 = jnp.sum(weight * weight, axis=-1)[None, :]
    wt = (-2.0 * weight.T).astype(jnp.bfloat16)
    kio = jnp.arange(KH, dtype=jnp.float32)[None, :]
    idx3, loss = _argmin_call(x, wt, x2, w2, kio)
    idx = idx3.reshape(N)
    weight_pad = jnp.pad(weight, ((0, 0), (0, 128 - D)))
    quantised = _sc_gather(weight_pad, idx.reshape(1, N))[:, :D]
    return (quantised, loss.reshape(()), idx)
